# Initial kernel scaffold; baseline (speedup 1.0000x reference)
import jax
import jax.numpy as jnp
from jax.experimental import pallas as pl


def kernel(target_x, target_edge_index, W1, b1, W2, b2, W3, b3, Wf1, bf1, Wf2, bf2):
    raise NotImplementedError("write your pallas kernel here")


# speedup vs baseline: 9.4269x; 9.4269x over previous
import functools
import math

import jax
import jax.numpy as jnp
from jax import lax
from jax.experimental import pallas as pl
from jax.experimental.pallas import tpu as pltpu
from jax.experimental.pallas import tpu_sc as plsc

N = 100000
E = 1600000
IN_DIM = 26

NC = 2
NS = 16
ROWS_PER_TILE = 6400
N_PAD = NS * ROWS_PER_TILE
EDGES_PER_SC = E // NC
EDGES_PER_TILE = EDGES_PER_SC // NS
EB = 1000
NBATCH = EDGES_PER_TILE // EB
ZROWS = 400

_MESH = dict(core_axis_name="c", subcore_axis_name="s")


_SC_PARAMS = pltpu.CompilerParams(use_tc_tiling_on_sc=False)


@functools.partial(
    pl.kernel,
    mesh=plsc.VectorSubcoreMesh(**_MESH),
    out_type=jax.ShapeDtypeStruct((NC, N_PAD, 16), jnp.float32),
    compiler_params=_SC_PARAMS,
    scratch_types=[
        pltpu.VMEM((EB,), jnp.int32),
        pltpu.VMEM((EB, 16), jnp.float32),
        pltpu.VMEM((ZROWS, 16), jnp.float32),
        pltpu.VMEM_SHARED((N_PAD, 16), jnp.float32),
    ],
)
def _deg_sc(dst_hbm, ones_hbm, zeros_hbm, out_hbm, dstv, onesv, zbuf, acc):
    c = lax.axis_index("c")
    s = lax.axis_index("s")
    row0 = s * ROWS_PER_TILE
    pltpu.sync_copy(zeros_hbm, zbuf)
    pltpu.sync_copy(ones_hbm, onesv)
    for k in range(ROWS_PER_TILE // ZROWS):
        pltpu.sync_copy(zbuf, acc.at[pl.ds(row0 + k * ZROWS, ZROWS)])
    plsc.subcore_barrier()
    ebase = c * EDGES_PER_SC + s * EDGES_PER_TILE

    def body(i, carry):
        pltpu.sync_copy(dst_hbm.at[pl.ds(ebase + i * EB, EB)], dstv)
        pltpu.sync_copy(onesv, acc.at[dstv], add=True)
        return carry

    lax.fori_loop(0, NBATCH, body, 0)
    plsc.subcore_barrier()
    pltpu.sync_copy(acc.at[pl.ds(row0, ROWS_PER_TILE)],
                    out_hbm.at[c, pl.ds(row0, ROWS_PER_TILE)])


def _make_scatter_sc(C):
    @functools.partial(
        pl.kernel,
        mesh=plsc.VectorSubcoreMesh(**_MESH),
        out_type=jax.ShapeDtypeStruct((NC, C, N_PAD, 16), jnp.float32),
        compiler_params=_SC_PARAMS,
        scratch_types=[
            pltpu.VMEM((EB,), jnp.int32),
            pltpu.VMEM((EB,), jnp.int32),
            pltpu.VMEM((EB, 16), jnp.float32),
            pltpu.VMEM((ZROWS, 16), jnp.float32),
            pltpu.VMEM_SHARED((N_PAD, 16), jnp.float32),
            pltpu.SemaphoreType.DMA,
        ],
    )
    def scatter_sc(src_hbm, dst_hbm, *rest):
        tables = rest[:C]
        zeros_hbm = rest[C]
        out_hbm = rest[C + 1]
        srcv, dstv, rowsv, zbuf, acc, sem = rest[C + 2:]
        c = lax.axis_index("c")
        s = lax.axis_index("s")
        row0 = s * ROWS_PER_TILE
        ebase = c * EDGES_PER_SC + s * EDGES_PER_TILE
        pltpu.sync_copy(zeros_hbm, zbuf)
        for k in range(ROWS_PER_TILE // ZROWS):
            pltpu.sync_copy(zbuf, acc.at[pl.ds(row0 + k * ZROWS, ZROWS)])
        plsc.subcore_barrier()
        for ch in range(C):
            tab = tables[ch]

            def body(i, carry):
                off = ebase + i * EB
                pltpu.sync_copy(src_hbm.at[pl.ds(off, EB)], srcv)
                pltpu.sync_copy(dst_hbm.at[pl.ds(off, EB)], dstv)
                pltpu.async_copy(tab.at[srcv], rowsv, sem).wait()
                pltpu.sync_copy(rowsv, acc.at[dstv], add=True)
                return carry

            lax.fori_loop(0, NBATCH, body, 0)
            plsc.subcore_barrier()
            pltpu.sync_copy(acc.at[pl.ds(row0, ROWS_PER_TILE)],
                            out_hbm.at[c, ch, pl.ds(row0, ROWS_PER_TILE)])
            for k in range(ROWS_PER_TILE // ZROWS):
                pltpu.sync_copy(zbuf, acc.at[pl.ds(row0 + k * ZROWS, ZROWS)])
            plsc.subcore_barrier()

    return scatter_sc


_BLK = 2000


def _t1(x, pe, deg0, deg1, W1p):
    C_out = W1p.shape[1] // 16
    grid = (N // _BLK,)

    def body(x_ref, pe_ref, d0_ref, d1_ref, w_ref, g_ref, dinv_ref):
        deg = d0_ref[...][:, :1] + d1_ref[...][:, :1] + 1.0
        dinv = 1.0 / jnp.sqrt(deg)
        xv = x_ref[...] + pe_ref[...]
        h = jnp.dot(xv, w_ref[...], preferred_element_type=jnp.float32)
        g = h * dinv
        for cch in range(C_out):
            g_ref[cch] = g[:, cch * 16:(cch + 1) * 16]
        dinv_ref[...] = dinv

    g1, dinv = pl.pallas_call(
        body,
        grid=grid,
        in_specs=[
            pl.BlockSpec((_BLK, IN_DIM), lambda i: (i, 0)),
            pl.BlockSpec((_BLK, IN_DIM), lambda i: (i, 0)),
            pl.BlockSpec((_BLK, 16), lambda i: (i, 0)),
            pl.BlockSpec((_BLK, 16), lambda i: (i, 0)),
            pl.BlockSpec(W1p.shape, lambda i: (0, 0)),
        ],
        out_specs=[
            pl.BlockSpec((C_out, _BLK, 16), lambda i: (0, i, 0)),
            pl.BlockSpec((_BLK, 1), lambda i: (i, 0)),
        ],
        out_shape=[
            jax.ShapeDtypeStruct((C_out, N, 16), jnp.float32),
            jax.ShapeDtypeStruct((N, 1), jnp.float32),
        ],
    )(x, pe, deg0, deg1, W1p)
    return g1, dinv


def _t2(p, gs, dinv, b_pad, W_pad):
    C_in = len(gs)
    C_out = W_pad.shape[1] // 16
    grid = (N // _BLK,)

    def body(*refs):
        p_ref = refs[0]
        g_refs = refs[1:1 + C_in]
        dinv_ref, b_ref, w_ref, out_ref = refs[1 + C_in:]
        dinv = dinv_ref[...]
        cols = []
        for cch in range(C_in):
            agg = p_ref[0, cch] + p_ref[1, cch] + g_refs[cch][...]
            cols.append(jnp.maximum(agg * dinv + b_ref[0, cch * 16:(cch + 1) * 16], 0.0))
        xt = jnp.concatenate(cols, axis=1)
        h = jnp.dot(xt, w_ref[...], preferred_element_type=jnp.float32)
        gn = h * dinv
        for cch in range(C_out):
            out_ref[cch] = gn[:, cch * 16:(cch + 1) * 16]

    out = pl.pallas_call(
        body,
        grid=grid,
        in_specs=[pl.BlockSpec((NC, C_in, _BLK, 16), lambda i: (0, 0, i, 0))]
        + [pl.BlockSpec((_BLK, 16), lambda i: (i, 0)) for _ in range(C_in)]
        + [
            pl.BlockSpec((_BLK, 1), lambda i: (i, 0)),
            pl.BlockSpec(b_pad.shape, lambda i: (0, 0)),
            pl.BlockSpec(W_pad.shape, lambda i: (0, 0)),
        ],
        out_specs=pl.BlockSpec((C_out, _BLK, 16), lambda i: (0, i, 0)),
        out_shape=jax.ShapeDtypeStruct((C_out, N, 16), jnp.float32),
    )(p, *gs, dinv, b_pad, W_pad)
    return [out[cch] for cch in range(C_out)]


def _t3(p, gs, dinv, b_pad, Wf1p, bf1, Wf2, bf2):
    C_in = len(gs)
    grid = (N // _BLK,)

    def body(*refs):
        p_ref = refs[0]
        g_refs = refs[1:1 + C_in]
        dinv_ref, b_ref, w1_ref, bf1_ref, w2_ref, bf2_ref, out_ref = refs[1 + C_in:]
        dinv = dinv_ref[...]
        cols = []
        for cch in range(C_in):
            agg = p_ref[0, cch] + p_ref[1, cch] + g_refs[cch][...]
            cols.append(jnp.maximum(agg * dinv + b_ref[0, cch * 16:(cch + 1) * 16], 0.0))
        xt = jnp.concatenate(cols, axis=1)
        t = jnp.dot(xt, w1_ref[...], preferred_element_type=jnp.float32)
        t = jnp.maximum(t + bf1_ref[...], 0.0)
        o = jnp.dot(t, w2_ref[...], preferred_element_type=jnp.float32)
        out_ref[...] = jnp.maximum(o + bf2_ref[...], 0.0)

    return pl.pallas_call(
        body,
        grid=grid,
        in_specs=[pl.BlockSpec((NC, C_in, _BLK, 16), lambda i: (0, 0, i, 0))]
        + [pl.BlockSpec((_BLK, 16), lambda i: (i, 0)) for _ in range(C_in)]
        + [
            pl.BlockSpec((_BLK, 1), lambda i: (i, 0)),
            pl.BlockSpec(b_pad.shape, lambda i: (0, 0)),
            pl.BlockSpec(Wf1p.shape, lambda i: (0, 0)),
            pl.BlockSpec((1, 1024), lambda i: (0, 0)),
            pl.BlockSpec(Wf2.shape, lambda i: (0, 0)),
            pl.BlockSpec((1, 128), lambda i: (0, 0)),
        ],
        out_specs=pl.BlockSpec((_BLK, 128), lambda i: (i, 0)),
        out_shape=jax.ShapeDtypeStruct((N, 128), jnp.float32),
    )(p, *gs, dinv, b_pad, Wf1p, bf1, Wf2, bf2)


def _pos_encoding(length, d_model):
    position = jnp.arange(length, dtype=jnp.float32)[:, None]
    div_term = jnp.exp(jnp.arange(0, d_model, 2).astype(jnp.float32)
                       * (-math.log(10000.0) / d_model))
    ang = position * div_term
    return jnp.stack([jnp.sin(ang), jnp.cos(ang)], axis=2).reshape(length, d_model)


def _pad2(w, rows, cols):
    out = jnp.zeros((rows, cols), jnp.float32)
    return out.at[: w.shape[0], : w.shape[1]].set(w)


_scatter2 = _make_scatter_sc(2)
_scatter4 = _make_scatter_sc(4)
_scatter7 = _make_scatter_sc(7)


def kernel(target_x, target_edge_index, W1, b1, W2, b2, W3, b3, Wf1, bf1, Wf2, bf2):
    ei = target_edge_index.astype(jnp.int32)
    src, dst = ei[0], ei[1]
    pe = _pos_encoding(N, IN_DIM)
    zeros16 = jnp.zeros((ZROWS, 16), jnp.float32)
    ones16 = jnp.ones((EB, 16), jnp.float32)

    W1p = _pad2(W1, IN_DIM, 32)
    b1p = _pad2(b1[None, :], 1, 32)
    W2p = _pad2(W2, 32, 64)
    b2p = _pad2(b2[None, :], 1, 64)
    W3p = _pad2(W3, 64, 112)
    b3p = _pad2(b3[None, :], 1, 112)
    Wf1p = _pad2(Wf1, 112, 1024)
    bf1r = bf1[None, :]
    bf2r = bf2[None, :]

    degp = _deg_sc(dst, ones16, zeros16)
    g1, dinv = _t1(target_x, pe, degp[0], degp[1], W1p)
    g1s = [g1[0], g1[1]]

    p1 = _scatter2(src, dst, *g1s, zeros16)
    g2s = _t2(p1, g1s, dinv, b1p, W2p)

    p2 = _scatter4(src, dst, *g2s, zeros16)
    g3s = _t2(p2, g2s, dinv, b2p, W3p)

    p3 = _scatter7(src, dst, *g3s, zeros16)
    out = _t3(p3, g3s, dinv, b3p, Wf1p, bf1r, Wf2, bf2r)
    return out[None]


# speedup vs baseline: 10.1182x; 1.0733x over previous
import functools
import math

import jax
import jax.numpy as jnp
from jax import lax
from jax.experimental import pallas as pl
from jax.experimental.pallas import tpu as pltpu
from jax.experimental.pallas import tpu_sc as plsc

N = 100000
E = 1600000
IN_DIM = 26

NC = 2
NS = 16
ROWS_PER_TILE = 6400
N_PAD = NS * ROWS_PER_TILE
EB = 640
NBATCH = 80
EDGES_PER_TILE = EB * NBATCH
EDGES_PER_SC = EDGES_PER_TILE * NS
E_PAD = EDGES_PER_SC * NC
ZROWS = 320

_MESH = dict(core_axis_name="c", subcore_axis_name="s")


_SC_PARAMS = pltpu.CompilerParams(use_tc_tiling_on_sc=False)


@functools.partial(
    pl.kernel,
    mesh=plsc.VectorSubcoreMesh(**_MESH),
    out_type=jax.ShapeDtypeStruct((NC, N_PAD, 16), jnp.float32),
    compiler_params=_SC_PARAMS,
    scratch_types=[
        pltpu.VMEM((EB,), jnp.int32),
        pltpu.VMEM((EB, 16), jnp.float32),
        pltpu.VMEM((ZROWS, 16), jnp.float32),
        pltpu.VMEM_SHARED((N_PAD, 16), jnp.float32),
    ],
)
def _deg_sc(dst_hbm, ones_hbm, zeros_hbm, out_hbm, dstv, onesv, zbuf, acc):
    c = lax.axis_index("c")
    s = lax.axis_index("s")
    row0 = s * ROWS_PER_TILE
    pltpu.sync_copy(zeros_hbm, zbuf)
    pltpu.sync_copy(ones_hbm, onesv)
    for k in range(ROWS_PER_TILE // ZROWS):
        pltpu.sync_copy(zbuf, acc.at[pl.ds(row0 + k * ZROWS, ZROWS)])
    plsc.subcore_barrier()
    ebase = c * EDGES_PER_SC + s * EDGES_PER_TILE

    def body(i, carry):
        pltpu.sync_copy(dst_hbm.at[pl.ds(ebase + i * EB, EB)], dstv)
        pltpu.sync_copy(onesv, acc.at[dstv], add=True)
        return carry

    lax.fori_loop(0, NBATCH, body, 0)
    plsc.subcore_barrier()
    pltpu.sync_copy(acc.at[pl.ds(row0, ROWS_PER_TILE)],
                    out_hbm.at[c, pl.ds(row0, ROWS_PER_TILE)])


def _make_scatter_sc(C):
    @functools.partial(
        pl.kernel,
        mesh=plsc.VectorSubcoreMesh(**_MESH),
        out_type=jax.ShapeDtypeStruct((NC, C, N_PAD, 16), jnp.float32),
        compiler_params=_SC_PARAMS,
        scratch_types=[
            pltpu.VMEM((EB,), jnp.int32),
            pltpu.VMEM((EB,), jnp.int32),
            pltpu.VMEM((EB,), jnp.int32),
            pltpu.VMEM((EB,), jnp.int32),
            pltpu.VMEM((EB, 16), jnp.float32),
            pltpu.VMEM((EB, 16), jnp.float32),
            pltpu.VMEM((ZROWS, 16), jnp.float32),
            pltpu.VMEM_SHARED((N_PAD, 16), jnp.float32),
            pltpu.SemaphoreType.DMA,
            pltpu.SemaphoreType.DMA,
        ],
    )
    def scatter_sc(src_hbm, dst_hbm, *rest):
        tables = rest[:C]
        zeros_hbm = rest[C]
        out_hbm = rest[C + 1]
        (src0, src1, dst0, dst1, rows0, rows1, zbuf, acc,
         sem0, sem1) = rest[C + 2:]
        c = lax.axis_index("c")
        s = lax.axis_index("s")
        row0 = s * ROWS_PER_TILE
        ebase = c * EDGES_PER_SC + s * EDGES_PER_TILE
        pltpu.sync_copy(zeros_hbm, zbuf)
        for k in range(ROWS_PER_TILE // ZROWS):
            pltpu.sync_copy(zbuf, acc.at[pl.ds(row0 + k * ZROWS, ZROWS)])
        plsc.subcore_barrier()

        def load_idx(j, sv, dv):
            off = ebase + j * EB
            pltpu.sync_copy(src_hbm.at[pl.ds(off, EB)], sv)
            pltpu.sync_copy(dst_hbm.at[pl.ds(off, EB)], dv)

        for ch in range(C):
            tab = tables[ch]

            load_idx(0, src0, dst0)
            pltpu.async_copy(tab.at[src0], rows0, sem0)
            load_idx(1, src1, dst1)
            pltpu.async_copy(tab.at[src1], rows1, sem1)

            def body(i, carry):
                j = 2 * i
                pltpu.make_async_copy(tab.at[src0], rows0, sem0).wait()
                pltpu.sync_copy(rows0, acc.at[dst0], add=True)
                load_idx(j + 2, src0, dst0)
                pltpu.async_copy(tab.at[src0], rows0, sem0)
                pltpu.make_async_copy(tab.at[src1], rows1, sem1).wait()
                pltpu.sync_copy(rows1, acc.at[dst1], add=True)
                load_idx(j + 3, src1, dst1)
                pltpu.async_copy(tab.at[src1], rows1, sem1)
                return carry

            lax.fori_loop(0, (NBATCH - 2) // 2, body, 0)
            pltpu.make_async_copy(tab.at[src0], rows0, sem0).wait()
            pltpu.sync_copy(rows0, acc.at[dst0], add=True)
            pltpu.make_async_copy(tab.at[src1], rows1, sem1).wait()
            pltpu.sync_copy(rows1, acc.at[dst1], add=True)

            plsc.subcore_barrier()
            pltpu.sync_copy(acc.at[pl.ds(row0, ROWS_PER_TILE)],
                            out_hbm.at[c, ch, pl.ds(row0, ROWS_PER_TILE)])
            for k in range(ROWS_PER_TILE // ZROWS):
                pltpu.sync_copy(zbuf, acc.at[pl.ds(row0 + k * ZROWS, ZROWS)])
            plsc.subcore_barrier()

    return scatter_sc


_BLK = 2048
_PBLK = _BLK // 8
NP2 = N_PAD // 8


def _t1(xp, degp, W1p):
    C_out = W1p.shape[1] // 16
    grid = (NP2 // _PBLK,)

    def body(x_ref, d_ref, w_ref, g_ref, dinv_ref):
        dinv = lax.rsqrt(d_ref[0] + d_ref[1] + 1.0)
        w = w_ref[...]
        hs = [jnp.dot(x_ref[:, j * IN_DIM:(j + 1) * IN_DIM], w,
                      preferred_element_type=jnp.float32)
              for j in range(8)]
        for cch in range(C_out):
            hp = jnp.concatenate(
                [h[:, cch * 16:(cch + 1) * 16] for h in hs], axis=1)
            g_ref[cch] = hp * dinv
        dinv_ref[...] = dinv

    g1, dinv = pl.pallas_call(
        body,
        grid=grid,
        in_specs=[
            pl.BlockSpec((_PBLK, 8 * IN_DIM), lambda i: (i, 0)),
            pl.BlockSpec((NC, _PBLK, 128), lambda i: (0, i, 0)),
            pl.BlockSpec(W1p.shape, lambda i: (0, 0)),
        ],
        out_specs=[
            pl.BlockSpec((C_out, _PBLK, 128), lambda i: (0, i, 0)),
            pl.BlockSpec((_PBLK, 128), lambda i: (i, 0)),
        ],
        out_shape=[
            jax.ShapeDtypeStruct((C_out, NP2, 128), jnp.float32),
            jax.ShapeDtypeStruct((NP2, 128), jnp.float32),
        ],
    )(xp, degp, W1p)
    return g1, dinv


def _t2(p, g, dinv, b_pack, W_pad):
    C_in = g.shape[0]
    C_out = W_pad.shape[1] // 16
    grid = (NP2 // _PBLK,)

    def body(p_ref, g_ref, dinv_ref, b_ref, w_ref, out_ref):
        dinv = dinv_ref[...]
        w = w_ref[...]
        xt = [jnp.maximum((p_ref[0, cch] + p_ref[1, cch] + g_ref[cch]) * dinv
                          + b_ref[cch], 0.0)
              for cch in range(C_in)]
        hs = []
        for j in range(8):
            xrow = jnp.concatenate(
                [xc[:, j * 16:(j + 1) * 16] for xc in xt], axis=1)
            hs.append(jnp.dot(xrow, w, preferred_element_type=jnp.float32))
        for cch in range(C_out):
            hp = jnp.concatenate(
                [h[:, cch * 16:(cch + 1) * 16] for h in hs], axis=1)
            out_ref[cch] = hp * dinv

    return pl.pallas_call(
        body,
        grid=grid,
        in_specs=[
            pl.BlockSpec((NC, C_in, _PBLK, 128), lambda i: (0, 0, i, 0)),
            pl.BlockSpec((C_in, _PBLK, 128), lambda i: (0, i, 0)),
            pl.BlockSpec((_PBLK, 128), lambda i: (i, 0)),
            pl.BlockSpec(b_pack.shape, lambda i: (0, 0, 0)),
            pl.BlockSpec(W_pad.shape, lambda i: (0, 0)),
        ],
        out_specs=pl.BlockSpec((C_out, _PBLK, 128), lambda i: (0, i, 0)),
        out_shape=jax.ShapeDtypeStruct((C_out, NP2, 128), jnp.float32),
    )(p, g, dinv, b_pack, W_pad)


def _t3(p, g, dinv, b_pack, Wf1p, bf1, Wf2, bf2):
    C_in = g.shape[0]
    grid = (NP2 // _PBLK,)

    def body(p_ref, g_ref, dinv_ref, b_ref, w1_ref, bf1_ref, w2_ref, bf2_ref,
             out_ref):
        dinv = dinv_ref[...]
        xt = [jnp.maximum((p_ref[0, cch] + p_ref[1, cch] + g_ref[cch]) * dinv
                          + b_ref[cch], 0.0)
              for cch in range(C_in)]
        w1 = w1_ref[...]
        w2 = w2_ref[...]
        for j in range(8):
            xrow = jnp.concatenate(
                [xc[:, j * 16:(j + 1) * 16] for xc in xt], axis=1)
            t = jnp.dot(xrow, w1, preferred_element_type=jnp.float32)
            t = jnp.maximum(t + bf1_ref[...], 0.0)
            o = jnp.dot(t, w2, preferred_element_type=jnp.float32)
            out_ref[:, j] = jnp.maximum(o + bf2_ref[...], 0.0)

    return pl.pallas_call(
        body,
        grid=grid,
        in_specs=[
            pl.BlockSpec((NC, C_in, _PBLK, 128), lambda i: (0, 0, i, 0)),
            pl.BlockSpec((C_in, _PBLK, 128), lambda i: (0, i, 0)),
            pl.BlockSpec((_PBLK, 128), lambda i: (i, 0)),
            pl.BlockSpec(b_pack.shape, lambda i: (0, 0, 0)),
            pl.BlockSpec(Wf1p.shape, lambda i: (0, 0)),
            pl.BlockSpec((1, 1024), lambda i: (0, 0)),
            pl.BlockSpec(Wf2.shape, lambda i: (0, 0)),
            pl.BlockSpec((1, 128), lambda i: (0, 0)),
        ],
        out_specs=pl.BlockSpec((_PBLK, 8, 128), lambda i: (i, 0, 0)),
        out_shape=jax.ShapeDtypeStruct((NP2, 8, 128), jnp.float32),
    )(p, g, dinv, b_pack, Wf1p, bf1, Wf2, bf2)


def _pos_encoding(length, d_model):
    position = jnp.arange(length, dtype=jnp.float32)[:, None]
    div_term = jnp.exp(jnp.arange(0, d_model, 2).astype(jnp.float32)
                       * (-math.log(10000.0) / d_model))
    ang = position * div_term
    return jnp.stack([jnp.sin(ang), jnp.cos(ang)], axis=2).reshape(length, d_model)


def _pad2(w, rows, cols):
    out = jnp.zeros((rows, cols), jnp.float32)
    return out.at[: w.shape[0], : w.shape[1]].set(w)


def _bias_pack(b, C):
    bp = _pad2(b[None, :], 1, C * 16).reshape(C, 1, 16)
    return jnp.tile(bp, (1, 1, 8)).reshape(C, 1, 128)


def _tables16(g_packed):
    return [g_packed[ch].reshape(N_PAD, 16) for ch in range(g_packed.shape[0])]


def _packedNP(p):
    return p.reshape(NC, p.shape[1], N_PAD // 8, 128)


_scatter2 = _make_scatter_sc(2)
_scatter4 = _make_scatter_sc(4)
_scatter7 = _make_scatter_sc(7)


def kernel(target_x, target_edge_index, W1, b1, W2, b2, W3, b3, Wf1, bf1, Wf2, bf2):
    ei = target_edge_index.astype(jnp.int32)
    epad = jnp.full((2, E_PAD - E), N, jnp.int32)
    src, dst = jnp.concatenate([ei, epad], axis=1)
    pe = _pos_encoding(N, IN_DIM)
    xv = jnp.zeros((N_PAD, IN_DIM), jnp.float32).at[:N].set(target_x + pe)
    xp = xv.reshape(NP2, 8 * IN_DIM)
    zeros16 = jnp.zeros((ZROWS, 16), jnp.float32)
    ones16 = jnp.ones((EB, 16), jnp.float32)

    W1p = _pad2(W1, IN_DIM, 32)
    W2p = _pad2(W2, 32, 64)
    W3p = _pad2(W3, 64, 112)
    Wf1p = _pad2(Wf1, 112, 1024)
    b1p = _bias_pack(b1, 2)
    b2p = _bias_pack(b2, 4)
    b3p = _bias_pack(b3, 7)
    bf1r = bf1[None, :]
    bf2r = bf2[None, :]

    degp = _deg_sc(dst, ones16, zeros16)
    g1, dinv = _t1(xp, degp.reshape(NC, N_PAD // 8, 128), W1p)

    p1 = _scatter2(src, dst, *_tables16(g1), zeros16)
    g2 = _t2(_packedNP(p1), g1, dinv, b1p, W2p)

    p2 = _scatter4(src, dst, *_tables16(g2), zeros16)
    g3 = _t2(_packedNP(p2), g2, dinv, b2p, W3p)

    p3 = _scatter7(src, dst, *_tables16(g3), zeros16)
    out = _t3(_packedNP(p3), g3, dinv, b3p, Wf1p, bf1r, Wf2, bf2r)
    return out.reshape(N_PAD, 128)[:N][None]


# speedup vs baseline: 10.3902x; 1.0269x over previous
import functools
import math

import jax
import jax.numpy as jnp
from jax import lax
from jax.experimental import pallas as pl
from jax.experimental.pallas import tpu as pltpu
from jax.experimental.pallas import tpu_sc as plsc

N = 100000
E = 1600000
IN_DIM = 26

NC = 2
NS = 16
ROWS_PER_TILE = 6400
N_PAD = NS * ROWS_PER_TILE
EB = 640
NBATCH = 80
EDGES_PER_TILE = EB * NBATCH
EDGES_PER_SC = EDGES_PER_TILE * NS
E_PAD = EDGES_PER_SC * NC
ZROWS = 320

_MESH = dict(core_axis_name="c", subcore_axis_name="s")


_SC_PARAMS = pltpu.CompilerParams(use_tc_tiling_on_sc=False)


@functools.partial(
    pl.kernel,
    mesh=plsc.VectorSubcoreMesh(**_MESH),
    out_type=jax.ShapeDtypeStruct((NC, N_PAD, 16), jnp.float32),
    compiler_params=_SC_PARAMS,
    scratch_types=[
        pltpu.VMEM((EB,), jnp.int32),
        pltpu.VMEM((EB, 16), jnp.float32),
        pltpu.VMEM((ZROWS, 16), jnp.float32),
        pltpu.VMEM_SHARED((N_PAD, 16), jnp.float32),
    ],
)
def _deg_sc(dst_hbm, ones_hbm, zeros_hbm, out_hbm, dstv, onesv, zbuf, acc):
    c = lax.axis_index("c")
    s = lax.axis_index("s")
    row0 = s * ROWS_PER_TILE
    pltpu.sync_copy(zeros_hbm, zbuf)
    pltpu.sync_copy(ones_hbm, onesv)
    for k in range(ROWS_PER_TILE // ZROWS):
        pltpu.sync_copy(zbuf, acc.at[pl.ds(row0 + k * ZROWS, ZROWS)])
    plsc.subcore_barrier()
    ebase = c * EDGES_PER_SC + s * EDGES_PER_TILE

    def body(i, carry):
        pltpu.sync_copy(dst_hbm.at[pl.ds(ebase + i * EB, EB)], dstv)
        pltpu.sync_copy(onesv, acc.at[dstv], add=True)
        return carry

    lax.fori_loop(0, NBATCH, body, 0)
    plsc.subcore_barrier()
    pltpu.sync_copy(acc.at[pl.ds(row0, ROWS_PER_TILE)],
                    out_hbm.at[c, pl.ds(row0, ROWS_PER_TILE)])


def _make_scatter_sc(C):
    @functools.partial(
        pl.kernel,
        mesh=plsc.VectorSubcoreMesh(**_MESH),
        out_type=jax.ShapeDtypeStruct((NC, C, N_PAD, 16), jnp.float32),
        compiler_params=_SC_PARAMS,
        scratch_types=[
            pltpu.VMEM((EB,), jnp.int32),
            pltpu.VMEM((EB,), jnp.int32),
            pltpu.VMEM((EB,), jnp.int32),
            pltpu.VMEM((EB,), jnp.int32),
            pltpu.VMEM((EB, 16), jnp.float32),
            pltpu.VMEM((EB, 16), jnp.float32),
            pltpu.VMEM((ZROWS, 16), jnp.float32),
            pltpu.VMEM_SHARED((N_PAD, 16), jnp.float32),
            pltpu.SemaphoreType.DMA,
            pltpu.SemaphoreType.DMA,
        ],
    )
    def scatter_sc(src_hbm, dst_hbm, *rest):
        tables = rest[:C]
        zeros_hbm = rest[C]
        out_hbm = rest[C + 1]
        (src0, src1, dst0, dst1, rows0, rows1, zbuf, acc,
         sem0, sem1) = rest[C + 2:]
        c = lax.axis_index("c")
        s = lax.axis_index("s")
        row0 = s * ROWS_PER_TILE
        ebase = c * EDGES_PER_SC + s * EDGES_PER_TILE
        pltpu.sync_copy(zeros_hbm, zbuf)
        for k in range(ROWS_PER_TILE // ZROWS):
            pltpu.sync_copy(zbuf, acc.at[pl.ds(row0 + k * ZROWS, ZROWS)])
        plsc.subcore_barrier()

        def load_idx(j, sv, dv):
            off = ebase + j * EB
            pltpu.sync_copy(src_hbm.at[pl.ds(off, EB)], sv)
            pltpu.sync_copy(dst_hbm.at[pl.ds(off, EB)], dv)

        for ch in range(C):
            tab = tables[ch]

            load_idx(0, src0, dst0)
            pltpu.async_copy(tab.at[src0], rows0, sem0)
            load_idx(1, src1, dst1)
            pltpu.async_copy(tab.at[src1], rows1, sem1)

            def body(i, carry):
                j = 2 * i
                pltpu.make_async_copy(tab.at[src0], rows0, sem0).wait()
                pltpu.sync_copy(rows0, acc.at[dst0], add=True)
                load_idx(j + 2, src0, dst0)
                pltpu.async_copy(tab.at[src0], rows0, sem0)
                pltpu.make_async_copy(tab.at[src1], rows1, sem1).wait()
                pltpu.sync_copy(rows1, acc.at[dst1], add=True)
                load_idx(j + 3, src1, dst1)
                pltpu.async_copy(tab.at[src1], rows1, sem1)
                return carry

            lax.fori_loop(0, (NBATCH - 2) // 2, body, 0)
            pltpu.make_async_copy(tab.at[src0], rows0, sem0).wait()
            pltpu.sync_copy(rows0, acc.at[dst0], add=True)
            pltpu.make_async_copy(tab.at[src1], rows1, sem1).wait()
            pltpu.sync_copy(rows1, acc.at[dst1], add=True)

            plsc.subcore_barrier()
            pltpu.sync_copy(acc.at[pl.ds(row0, ROWS_PER_TILE)],
                            out_hbm.at[c, ch, pl.ds(row0, ROWS_PER_TILE)])
            for k in range(ROWS_PER_TILE // ZROWS):
                pltpu.sync_copy(zbuf, acc.at[pl.ds(row0 + k * ZROWS, ZROWS)])
            plsc.subcore_barrier()

    return scatter_sc


_BLK = 2048
_PBLK = _BLK // 8
NP2 = N_PAD // 8


def _t1(xp, degp, W1p):
    C_out = W1p.shape[1] // 16
    grid = (NP2 // _PBLK,)

    def body(x_ref, d_ref, w_ref, g_ref, dinv_ref):
        dinv = lax.rsqrt(d_ref[0] + d_ref[1] + 1.0)
        w = w_ref[...]
        hs = [jnp.dot(x_ref[:, j * IN_DIM:(j + 1) * IN_DIM], w,
                      preferred_element_type=jnp.float32)
              for j in range(8)]
        for cch in range(C_out):
            hp = jnp.concatenate(
                [h[:, cch * 16:(cch + 1) * 16] for h in hs], axis=1)
            g_ref[cch] = hp * dinv
        dinv_ref[...] = dinv

    g1, dinv = pl.pallas_call(
        body,
        grid=grid,
        in_specs=[
            pl.BlockSpec((_PBLK, 8 * IN_DIM), lambda i: (i, 0)),
            pl.BlockSpec((NC, _PBLK, 128), lambda i: (0, i, 0)),
            pl.BlockSpec(W1p.shape, lambda i: (0, 0)),
        ],
        out_specs=[
            pl.BlockSpec((C_out, _PBLK, 128), lambda i: (0, i, 0)),
            pl.BlockSpec((_PBLK, 128), lambda i: (i, 0)),
        ],
        out_shape=[
            jax.ShapeDtypeStruct((C_out, NP2, 128), jnp.float32),
            jax.ShapeDtypeStruct((NP2, 128), jnp.float32),
        ],
    )(xp, degp, W1p)
    return g1, dinv


def _t2(p, g, dinv, b_pack, W_pad):
    C_in = g.shape[0]
    C_out = W_pad.shape[1] // 16
    grid = (NP2 // _PBLK,)

    def body(p_ref, g_ref, dinv_ref, b_ref, w_ref, out_ref):
        dinv = dinv_ref[...]
        w = w_ref[...]
        xt = [jnp.maximum((p_ref[0, cch] + p_ref[1, cch] + g_ref[cch]) * dinv
                          + b_ref[cch], 0.0)
              for cch in range(C_in)]
        hs = []
        for j in range(8):
            xrow = jnp.concatenate(
                [xc[:, j * 16:(j + 1) * 16] for xc in xt], axis=1)
            hs.append(jnp.dot(xrow, w, preferred_element_type=jnp.float32))
        for cch in range(C_out):
            hp = jnp.concatenate(
                [h[:, cch * 16:(cch + 1) * 16] for h in hs], axis=1)
            out_ref[cch] = hp * dinv

    return pl.pallas_call(
        body,
        grid=grid,
        in_specs=[
            pl.BlockSpec((NC, C_in, _PBLK, 128), lambda i: (0, 0, i, 0)),
            pl.BlockSpec((C_in, _PBLK, 128), lambda i: (0, i, 0)),
            pl.BlockSpec((_PBLK, 128), lambda i: (i, 0)),
            pl.BlockSpec(b_pack.shape, lambda i: (0, 0, 0)),
            pl.BlockSpec(W_pad.shape, lambda i: (0, 0)),
        ],
        out_specs=pl.BlockSpec((C_out, _PBLK, 128), lambda i: (0, i, 0)),
        out_shape=jax.ShapeDtypeStruct((C_out, NP2, 128), jnp.float32),
    )(p, g, dinv, b_pack, W_pad)


def _t3(p, g, dinv, b_pack, Wf1p, bf1, Wf2, bf2):
    C_in = g.shape[0]
    grid = (NP2 // _PBLK,)

    def body(p_ref, g_ref, dinv_ref, b_ref, w1_ref, bf1_ref, w2_ref, bf2_ref,
             out_ref):
        dinv = dinv_ref[...]
        xt = [jnp.maximum((p_ref[0, cch] + p_ref[1, cch] + g_ref[cch]) * dinv
                          + b_ref[cch], 0.0)
              for cch in range(C_in)]
        w1 = w1_ref[...]
        w2 = w2_ref[...]
        for j in range(8):
            xrow = jnp.concatenate(
                [xc[:, j * 16:(j + 1) * 16] for xc in xt], axis=1)
            t = jnp.dot(xrow, w1, preferred_element_type=jnp.float32)
            t = jnp.maximum(t + bf1_ref[...], 0.0)
            o = jnp.dot(t, w2, preferred_element_type=jnp.float32)
            out_ref[:, j] = jnp.maximum(o + bf2_ref[...], 0.0)

    return pl.pallas_call(
        body,
        grid=grid,
        in_specs=[
            pl.BlockSpec((NC, C_in, _PBLK, 128), lambda i: (0, 0, i, 0)),
            pl.BlockSpec((C_in, _PBLK, 128), lambda i: (0, i, 0)),
            pl.BlockSpec((_PBLK, 128), lambda i: (i, 0)),
            pl.BlockSpec(b_pack.shape, lambda i: (0, 0, 0)),
            pl.BlockSpec(Wf1p.shape, lambda i: (0, 0)),
            pl.BlockSpec((1, 1024), lambda i: (0, 0)),
            pl.BlockSpec(Wf2.shape, lambda i: (0, 0)),
            pl.BlockSpec((1, 128), lambda i: (0, 0)),
        ],
        out_specs=pl.BlockSpec((_PBLK, 8, 128), lambda i: (i, 0, 0)),
        out_shape=jax.ShapeDtypeStruct((NP2, 8, 128), jnp.float32),
    )(p, g, dinv, b_pack, Wf1p, bf1, Wf2, bf2)


def _pos_encoding(length, d_model):
    position = jnp.arange(length, dtype=jnp.float32)[:, None]
    div_term = jnp.exp(jnp.arange(0, d_model, 2).astype(jnp.float32)
                       * (-math.log(10000.0) / d_model))
    ang = position * div_term
    return jnp.stack([jnp.sin(ang), jnp.cos(ang)], axis=2).reshape(length, d_model)


def _pad2(w, rows, cols):
    out = jnp.zeros((rows, cols), jnp.float32)
    return out.at[: w.shape[0], : w.shape[1]].set(w)


def _bias_pack(b, C):
    bp = _pad2(b[None, :], 1, C * 16).reshape(C, 1, 16)
    return jnp.tile(bp, (1, 1, 8)).reshape(C, 1, 128)


def _tables16(g_packed):
    return [g_packed[ch].reshape(N_PAD, 16) for ch in range(g_packed.shape[0])]


def _packedNP(p):
    return p.reshape(NC, p.shape[1], N_PAD // 8, 128)


_scatter2 = _make_scatter_sc(2)
_scatter4 = _make_scatter_sc(4)
_scatter7 = _make_scatter_sc(7)


def kernel(target_x, target_edge_index, W1, b1, W2, b2, W3, b3, Wf1, bf1, Wf2, bf2):
    ei = target_edge_index.astype(jnp.int32)
    ndum = E_PAD - E
    dum_src = jnp.full((ndum,), N, jnp.int32)
    dum_dst = N + jnp.arange(ndum, dtype=jnp.int32) % (N_PAD - N)
    src = jnp.concatenate([ei[0], dum_src])
    dst = jnp.concatenate([ei[1], dum_dst])
    pe = _pos_encoding(N, IN_DIM)
    xv = jnp.zeros((N_PAD, IN_DIM), jnp.float32).at[:N].set(target_x + pe)
    xp = xv.reshape(NP2, 8 * IN_DIM)
    zeros16 = jnp.zeros((ZROWS, 16), jnp.float32)
    ones16 = jnp.ones((EB, 16), jnp.float32)

    W1p = _pad2(W1, IN_DIM, 32)
    W2p = _pad2(W2, 32, 64)
    W3p = _pad2(W3, 64, 112)
    Wf1p = _pad2(Wf1, 112, 1024)
    b1p = _bias_pack(b1, 2)
    b2p = _bias_pack(b2, 4)
    b3p = _bias_pack(b3, 7)
    bf1r = bf1[None, :]
    bf2r = bf2[None, :]

    degp = _deg_sc(dst, ones16, zeros16)
    g1, dinv = _t1(xp, degp.reshape(NC, N_PAD // 8, 128), W1p)

    p1 = _scatter2(src, dst, *_tables16(g1), zeros16)
    g2 = _t2(_packedNP(p1), g1, dinv, b1p, W2p)

    p2 = _scatter4(src, dst, *_tables16(g2), zeros16)
    g3 = _t2(_packedNP(p2), g2, dinv, b2p, W3p)

    p3 = _scatter7(src, dst, *_tables16(g3), zeros16)
    out = _t3(_packedNP(p3), g3, dinv, b3p, Wf1p, bf1r, Wf2, bf2r)
    return out.reshape(N_PAD, 128)[:N][None]


# speedup vs baseline: 18.8063x; 1.8100x over previous
import functools
import math

import jax
import jax.numpy as jnp
from jax import lax
from jax.experimental import pallas as pl
from jax.experimental.pallas import tpu as pltpu
from jax.experimental.pallas import tpu_sc as plsc

N = 100000
E = 1600000
IN_DIM = 26

NC = 2
NS = 16
ROWS_PER_TILE = 6400
N_PAD = NS * ROWS_PER_TILE
EB = 640
NBATCH = 80
EDGES_PER_TILE = EB * NBATCH
EDGES_PER_SC = EDGES_PER_TILE * NS
E_PAD = EDGES_PER_SC * NC
ZROWS = 320

_MESH = dict(core_axis_name="c", subcore_axis_name="s")


_SC_PARAMS = pltpu.CompilerParams(use_tc_tiling_on_sc=False)


@functools.partial(
    pl.kernel,
    mesh=plsc.VectorSubcoreMesh(**_MESH),
    out_type=jax.ShapeDtypeStruct((NC, N_PAD, 16), jnp.float32),
    compiler_params=_SC_PARAMS,
    scratch_types=[
        pltpu.VMEM((EB,), jnp.int32),
        pltpu.VMEM((EB, 16), jnp.float32),
        pltpu.VMEM((ZROWS, 16), jnp.float32),
        pltpu.VMEM_SHARED((N_PAD, 16), jnp.float32),
    ],
)
def _deg_sc(dst_hbm, ones_hbm, zeros_hbm, out_hbm, dstv, onesv, zbuf, acc):
    c = lax.axis_index("c")
    s = lax.axis_index("s")
    row0 = s * ROWS_PER_TILE
    pltpu.sync_copy(zeros_hbm, zbuf)
    pltpu.sync_copy(ones_hbm, onesv)
    for k in range(ROWS_PER_TILE // ZROWS):
        pltpu.sync_copy(zbuf, acc.at[pl.ds(row0 + k * ZROWS, ZROWS)])
    plsc.subcore_barrier()
    ebase = c * EDGES_PER_SC + s * EDGES_PER_TILE

    def body(i, carry):
        pltpu.sync_copy(dst_hbm.at[pl.ds(ebase + i * EB, EB)], dstv)
        pltpu.sync_copy(onesv, acc.at[dstv], add=True)
        return carry

    lax.fori_loop(0, NBATCH, body, 0)
    plsc.subcore_barrier()
    pltpu.sync_copy(acc.at[pl.ds(row0, ROWS_PER_TILE)],
                    out_hbm.at[c, pl.ds(row0, ROWS_PER_TILE)])


def _make_scatter_sc(C):
    @functools.partial(
        pl.kernel,
        mesh=plsc.VectorSubcoreMesh(**_MESH),
        out_type=jax.ShapeDtypeStruct((NC, C, N_PAD, 16), jnp.float32),
        compiler_params=_SC_PARAMS,
        scratch_types=[
            pltpu.VMEM((EB,), jnp.int32),
            pltpu.VMEM((EB,), jnp.int32),
            pltpu.VMEM((EB,), jnp.int32),
            pltpu.VMEM((EB,), jnp.int32),
            pltpu.VMEM((EB, 16), jnp.float32),
            pltpu.VMEM((EB, 16), jnp.float32),
            pltpu.VMEM((ZROWS, 16), jnp.float32),
            pltpu.VMEM_SHARED((N_PAD, 16), jnp.float32),
            pltpu.SemaphoreType.DMA,
            pltpu.SemaphoreType.DMA,
        ],
    )
    def scatter_sc(src_hbm, dst_hbm, *rest):
        tables = rest[:C]
        zeros_hbm = rest[C]
        out_hbm = rest[C + 1]
        (src0, src1, dst0, dst1, rows0, rows1, zbuf, acc,
         sem0, sem1) = rest[C + 2:]
        c = lax.axis_index("c")
        s = lax.axis_index("s")
        row0 = s * ROWS_PER_TILE
        ebase = c * EDGES_PER_SC + s * EDGES_PER_TILE
        pltpu.sync_copy(zeros_hbm, zbuf)
        for k in range(ROWS_PER_TILE // ZROWS):
            pltpu.sync_copy(zbuf, acc.at[pl.ds(row0 + k * ZROWS, ZROWS)])
        plsc.subcore_barrier()

        def load_idx(j, sv, dv):
            off = ebase + j * EB
            pltpu.sync_copy(src_hbm.at[pl.ds(off, EB)], sv)
            pltpu.sync_copy(dst_hbm.at[pl.ds(off, EB)], dv)

        for ch in range(C):
            tab = tables[ch]

            load_idx(0, src0, dst0)
            pltpu.async_copy(tab.at[src0], rows0, sem0)
            load_idx(1, src1, dst1)
            pltpu.async_copy(tab.at[src1], rows1, sem1)

            def body(i, carry):
                j = 2 * i
                pltpu.make_async_copy(tab.at[src0], rows0, sem0).wait()
                pltpu.sync_copy(rows0, acc.at[dst0], add=True)
                load_idx(j + 2, src0, dst0)
                pltpu.async_copy(tab.at[src0], rows0, sem0)
                pltpu.make_async_copy(tab.at[src1], rows1, sem1).wait()
                pltpu.sync_copy(rows1, acc.at[dst1], add=True)
                load_idx(j + 3, src1, dst1)
                pltpu.async_copy(tab.at[src1], rows1, sem1)
                return carry

            lax.fori_loop(0, (NBATCH - 2) // 2, body, 0)
            pltpu.make_async_copy(tab.at[src0], rows0, sem0).wait()
            pltpu.sync_copy(rows0, acc.at[dst0], add=True)
            pltpu.make_async_copy(tab.at[src1], rows1, sem1).wait()
            pltpu.sync_copy(rows1, acc.at[dst1], add=True)

            plsc.subcore_barrier()
            pltpu.sync_copy(acc.at[pl.ds(row0, ROWS_PER_TILE)],
                            out_hbm.at[c, ch, pl.ds(row0, ROWS_PER_TILE)])
            for k in range(ROWS_PER_TILE // ZROWS):
                pltpu.sync_copy(zbuf, acc.at[pl.ds(row0 + k * ZROWS, ZROWS)])
            plsc.subcore_barrier()

    return scatter_sc


_BLK = 2048
_PBLK = _BLK // 8
NP2 = N_PAD // 8


def _t1(xp, degp, W1p):
    C_out = W1p.shape[1] // 16
    grid = (NP2 // _PBLK,)

    def body(x_ref, d_ref, w_ref, g_ref, dinv_ref):
        dinv = lax.rsqrt(d_ref[0] + d_ref[1] + 1.0)
        w = w_ref[...]
        hs = [jnp.dot(x_ref[:, j * IN_DIM:(j + 1) * IN_DIM], w,
                      preferred_element_type=jnp.float32)
              for j in range(8)]
        for cch in range(C_out):
            hp = jnp.concatenate(
                [h[:, cch * 16:(cch + 1) * 16] for h in hs], axis=1)
            g_ref[cch] = hp * dinv
        dinv_ref[...] = dinv

    g1, dinv = pl.pallas_call(
        body,
        grid=grid,
        in_specs=[
            pl.BlockSpec((_PBLK, 8 * IN_DIM), lambda i: (i, 0)),
            pl.BlockSpec((NC, _PBLK, 128), lambda i: (0, i, 0)),
            pl.BlockSpec(W1p.shape, lambda i: (0, 0)),
        ],
        out_specs=[
            pl.BlockSpec((C_out, _PBLK, 128), lambda i: (0, i, 0)),
            pl.BlockSpec((_PBLK, 128), lambda i: (i, 0)),
        ],
        out_shape=[
            jax.ShapeDtypeStruct((C_out, NP2, 128), jnp.float32),
            jax.ShapeDtypeStruct((NP2, 128), jnp.float32),
        ],
    )(xp, degp, W1p)
    return g1, dinv


def _t2(p, g, dinv, b_pack, W_pad):
    C_in = g.shape[0]
    C_out = W_pad.shape[1] // 16
    grid = (NP2 // _PBLK,)

    def body(p_ref, g_ref, dinv_ref, b_ref, w_ref, out_ref):
        dinv = dinv_ref[...]
        w = w_ref[...]
        xt = [jnp.maximum((p_ref[0, cch] + p_ref[1, cch] + g_ref[cch]) * dinv
                          + b_ref[cch], 0.0)
              for cch in range(C_in)]
        hs = []
        for j in range(8):
            xrow = jnp.concatenate(
                [xc[:, j * 16:(j + 1) * 16] for xc in xt], axis=1)
            hs.append(jnp.dot(xrow, w, preferred_element_type=jnp.float32))
        for cch in range(C_out):
            hp = jnp.concatenate(
                [h[:, cch * 16:(cch + 1) * 16] for h in hs], axis=1)
            out_ref[cch] = hp * dinv

    return pl.pallas_call(
        body,
        grid=grid,
        in_specs=[
            pl.BlockSpec((NC, C_in, _PBLK, 128), lambda i: (0, 0, i, 0)),
            pl.BlockSpec((C_in, _PBLK, 128), lambda i: (0, i, 0)),
            pl.BlockSpec((_PBLK, 128), lambda i: (i, 0)),
            pl.BlockSpec(b_pack.shape, lambda i: (0, 0, 0)),
            pl.BlockSpec(W_pad.shape, lambda i: (0, 0)),
        ],
        out_specs=pl.BlockSpec((C_out, _PBLK, 128), lambda i: (0, i, 0)),
        out_shape=jax.ShapeDtypeStruct((C_out, NP2, 128), jnp.float32),
    )(p, g, dinv, b_pack, W_pad)


def _t3(p, g, dinv, b_pack, Wf1p, bf1, Wf2, bf2):
    C_in = g.shape[0]
    grid = (NP2 // _PBLK,)

    def body(p_ref, g_ref, dinv_ref, b_ref, w1_ref, bf1_ref, w2_ref, bf2_ref,
             out_ref):
        dinv = dinv_ref[...]
        xt = [jnp.maximum((p_ref[0, cch] + p_ref[1, cch] + g_ref[cch]) * dinv
                          + b_ref[cch], 0.0)
              for cch in range(C_in)]
        w1 = w1_ref[...]
        w2 = w2_ref[...]
        for j in range(8):
            xrow = jnp.concatenate(
                [xc[:, j * 16:(j + 1) * 16] for xc in xt], axis=1)
            t = jnp.dot(xrow, w1, preferred_element_type=jnp.float32)
            t = jnp.maximum(t + bf1_ref[...], 0.0)
            o = jnp.dot(t, w2, preferred_element_type=jnp.float32)
            out_ref[:, j] = jnp.maximum(o + bf2_ref[...], 0.0)

    return pl.pallas_call(
        body,
        grid=grid,
        in_specs=[
            pl.BlockSpec((NC, C_in, _PBLK, 128), lambda i: (0, 0, i, 0)),
            pl.BlockSpec((C_in, _PBLK, 128), lambda i: (0, i, 0)),
            pl.BlockSpec((_PBLK, 128), lambda i: (i, 0)),
            pl.BlockSpec(b_pack.shape, lambda i: (0, 0, 0)),
            pl.BlockSpec(Wf1p.shape, lambda i: (0, 0)),
            pl.BlockSpec((1, 1024), lambda i: (0, 0)),
            pl.BlockSpec(Wf2.shape, lambda i: (0, 0)),
            pl.BlockSpec((1, 128), lambda i: (0, 0)),
        ],
        out_specs=pl.BlockSpec((_PBLK, 8, 128), lambda i: (i, 0, 0)),
        out_shape=jax.ShapeDtypeStruct((NP2, 8, 128), jnp.float32),
    )(p, g, dinv, b_pack, Wf1p, bf1, Wf2, bf2)


def _pos_encoding(length, d_model):
    position = jnp.arange(length, dtype=jnp.float32)[:, None]
    div_term = jnp.exp(jnp.arange(0, d_model, 2).astype(jnp.float32)
                       * (-math.log(10000.0) / d_model))
    ang = position * div_term
    return jnp.stack([jnp.sin(ang), jnp.cos(ang)], axis=2).reshape(length, d_model)


def _pad2(w, rows, cols):
    out = jnp.zeros((rows, cols), jnp.float32)
    return out.at[: w.shape[0], : w.shape[1]].set(w)


def _bias_pack(b, C):
    bp = _pad2(b[None, :], 1, C * 16).reshape(C, 1, 16)
    return jnp.tile(bp, (1, 1, 8)).reshape(C, 1, 128)


def _tables16(g_packed):
    return [g_packed[ch].reshape(N_PAD, 16) for ch in range(g_packed.shape[0])]


def _packedNP(p):
    return p.reshape(NC, p.shape[1], N_PAD // 8, 128)


_scatter2 = _make_scatter_sc(2)
_scatter4 = _make_scatter_sc(4)
_scatter7 = _make_scatter_sc(7)


def kernel(target_x, target_edge_index, W1, b1, W2, b2, W3, b3, Wf1, bf1, Wf2, bf2):
    ei = target_edge_index.astype(jnp.int32)
    ndum = E_PAD - E
    dum = N + jnp.arange(ndum, dtype=jnp.int32) % (N_PAD - N)
    src = jnp.concatenate([ei[0], dum])
    dst = jnp.concatenate([ei[1], dum])
    pe = _pos_encoding(N, IN_DIM)
    xv = jnp.zeros((N_PAD, IN_DIM), jnp.float32).at[:N].set(target_x + pe)
    xp = xv.reshape(NP2, 8 * IN_DIM)
    zeros16 = jnp.zeros((ZROWS, 16), jnp.float32)
    ones16 = jnp.ones((EB, 16), jnp.float32)

    W1p = _pad2(W1, IN_DIM, 32)
    W2p = _pad2(W2, 32, 64)
    W3p = _pad2(W3, 64, 112)
    Wf1p = _pad2(Wf1, 112, 1024)
    b1p = _bias_pack(b1, 2)
    b2p = _bias_pack(b2, 4)
    b3p = _bias_pack(b3, 7)
    bf1r = bf1[None, :]
    bf2r = bf2[None, :]

    degp = _deg_sc(dst, ones16, zeros16)
    g1, dinv = _t1(xp, degp.reshape(NC, N_PAD // 8, 128), W1p)

    p1 = _scatter2(src, dst, *_tables16(g1), zeros16)
    g2 = _t2(_packedNP(p1), g1, dinv, b1p, W2p)

    p2 = _scatter4(src, dst, *_tables16(g2), zeros16)
    g3 = _t2(_packedNP(p2), g2, dinv, b2p, W3p)

    p3 = _scatter7(src, dst, *_tables16(g3), zeros16)
    out = _t3(_packedNP(p3), g3, dinv, b3p, Wf1p, bf1r, Wf2, bf2r)
    return out.reshape(N_PAD, 128)[:N][None]


# speedup vs baseline: 21.4177x; 1.1389x over previous
import functools
import math

import jax
import jax.numpy as jnp
from jax import lax
from jax.experimental import pallas as pl
from jax.experimental.pallas import tpu as pltpu
from jax.experimental.pallas import tpu_sc as plsc

N = 100000
E = 1600000
IN_DIM = 26

NC = 2
NS = 16
ROWS_PER_TILE = 6400
N_PAD = NS * ROWS_PER_TILE
EB = 640
NBATCH = 80
EDGES_PER_TILE = EB * NBATCH
EDGES_PER_SC = EDGES_PER_TILE * NS
E_PAD = EDGES_PER_SC * NC
ZROWS = 320

_MESH = dict(core_axis_name="c", subcore_axis_name="s")


_SC_PARAMS = pltpu.CompilerParams(use_tc_tiling_on_sc=False)


@functools.partial(
    pl.kernel,
    mesh=plsc.VectorSubcoreMesh(**_MESH),
    out_type=jax.ShapeDtypeStruct((NC, N_PAD, 16), jnp.float32),
    compiler_params=_SC_PARAMS,
    scratch_types=[
        pltpu.VMEM((EB,), jnp.int32),
        pltpu.VMEM((EB, 16), jnp.float32),
        pltpu.VMEM((ZROWS, 16), jnp.float32),
        pltpu.VMEM_SHARED((N_PAD, 16), jnp.float32),
    ],
)
def _deg_sc(dst_hbm, ones_hbm, zeros_hbm, out_hbm, dstv, onesv, zbuf, acc):
    c = lax.axis_index("c")
    s = lax.axis_index("s")
    row0 = s * ROWS_PER_TILE
    pltpu.sync_copy(zeros_hbm, zbuf)
    pltpu.sync_copy(ones_hbm, onesv)
    for k in range(ROWS_PER_TILE // ZROWS):
        pltpu.sync_copy(zbuf, acc.at[pl.ds(row0 + k * ZROWS, ZROWS)])
    plsc.subcore_barrier()
    ebase = c * EDGES_PER_SC + s * EDGES_PER_TILE

    def body(i, carry):
        pltpu.sync_copy(dst_hbm.at[pl.ds(ebase + i * EB, EB)], dstv)
        pltpu.sync_copy(onesv, acc.at[dstv], add=True)
        return carry

    lax.fori_loop(0, NBATCH, body, 0)
    plsc.subcore_barrier()
    pltpu.sync_copy(acc.at[pl.ds(row0, ROWS_PER_TILE)],
                    out_hbm.at[c, pl.ds(row0, ROWS_PER_TILE)])


def _make_scatter_sc(C):
    @functools.partial(
        pl.kernel,
        mesh=plsc.VectorSubcoreMesh(**_MESH),
        out_type=jax.ShapeDtypeStruct((NC, C, N_PAD, 16), jnp.float32),
        compiler_params=_SC_PARAMS,
        scratch_types=[
            pltpu.VMEM((2, EB), jnp.int32),
            pltpu.VMEM((2, EB), jnp.int32),
            pltpu.VMEM((EB, 16), jnp.float32),
            pltpu.VMEM((EB, 16), jnp.float32),
            pltpu.VMEM((ZROWS, 16), jnp.float32),
            pltpu.VMEM_SHARED((N_PAD, 16), jnp.float32),
            pltpu.SemaphoreType.DMA,
            pltpu.SemaphoreType.DMA,
        ],
    )
    def scatter_sc(idx_hbm, *rest):
        tables = rest[:C]
        zeros_hbm = rest[C]
        out_hbm = rest[C + 1]
        (idx0, idx1, rows0, rows1, zbuf, acc, sem0, sem1) = rest[C + 2:]
        c = lax.axis_index("c")
        s = lax.axis_index("s")
        row0 = s * ROWS_PER_TILE
        bbase = (c * NS + s) * NBATCH
        pltpu.sync_copy(zeros_hbm, zbuf)
        for k in range(ROWS_PER_TILE // ZROWS):
            pltpu.sync_copy(zbuf, acc.at[pl.ds(row0 + k * ZROWS, ZROWS)])
        plsc.subcore_barrier()

        for ch in range(C):
            tab = tables[ch]

            pltpu.sync_copy(idx_hbm.at[bbase], idx0)
            pltpu.async_copy(tab.at[idx0.at[0]], rows0, sem0)
            pltpu.sync_copy(idx_hbm.at[bbase + 1], idx1)
            pltpu.async_copy(tab.at[idx1.at[0]], rows1, sem1)

            def body(i, carry):
                j = bbase + 2 * i
                pltpu.make_async_copy(tab.at[idx0.at[0]], rows0, sem0).wait()
                pltpu.sync_copy(rows0, acc.at[idx0.at[1]], add=True)
                pltpu.sync_copy(idx_hbm.at[j + 2], idx0)
                pltpu.async_copy(tab.at[idx0.at[0]], rows0, sem0)
                pltpu.make_async_copy(tab.at[idx1.at[0]], rows1, sem1).wait()
                pltpu.sync_copy(rows1, acc.at[idx1.at[1]], add=True)
                pltpu.sync_copy(idx_hbm.at[j + 3], idx1)
                pltpu.async_copy(tab.at[idx1.at[0]], rows1, sem1)
                return carry

            lax.fori_loop(0, (NBATCH - 2) // 2, body, 0)
            pltpu.make_async_copy(tab.at[idx0.at[0]], rows0, sem0).wait()
            pltpu.sync_copy(rows0, acc.at[idx0.at[1]], add=True)
            pltpu.make_async_copy(tab.at[idx1.at[0]], rows1, sem1).wait()
            pltpu.sync_copy(rows1, acc.at[idx1.at[1]], add=True)

            plsc.subcore_barrier()
            pltpu.sync_copy(acc.at[pl.ds(row0, ROWS_PER_TILE)],
                            out_hbm.at[c, ch, pl.ds(row0, ROWS_PER_TILE)])
            for k in range(ROWS_PER_TILE // ZROWS):
                pltpu.sync_copy(zbuf, acc.at[pl.ds(row0 + k * ZROWS, ZROWS)])
            plsc.subcore_barrier()

    return scatter_sc


_BLK = 2048
_PBLK = _BLK // 8
NP2 = N_PAD // 8


def _t1(xp, degp, W1p):
    C_out = W1p.shape[1] // 16
    grid = (NP2 // _PBLK,)

    def body(x_ref, d_ref, w_ref, g_ref, dinv_ref):
        dinv = lax.rsqrt(d_ref[0] + d_ref[1] + 1.0)
        w = w_ref[...]
        hs = [jnp.dot(x_ref[:, j * IN_DIM:(j + 1) * IN_DIM], w,
                      preferred_element_type=jnp.float32)
              for j in range(8)]
        for cch in range(C_out):
            hp = jnp.concatenate(
                [h[:, cch * 16:(cch + 1) * 16] for h in hs], axis=1)
            g_ref[cch] = hp * dinv
        dinv_ref[...] = dinv

    g1, dinv = pl.pallas_call(
        body,
        grid=grid,
        in_specs=[
            pl.BlockSpec((_PBLK, 8 * IN_DIM), lambda i: (i, 0)),
            pl.BlockSpec((NC, _PBLK, 128), lambda i: (0, i, 0)),
            pl.BlockSpec(W1p.shape, lambda i: (0, 0)),
        ],
        out_specs=[
            pl.BlockSpec((C_out, _PBLK, 128), lambda i: (0, i, 0)),
            pl.BlockSpec((_PBLK, 128), lambda i: (i, 0)),
        ],
        out_shape=[
            jax.ShapeDtypeStruct((C_out, NP2, 128), jnp.float32),
            jax.ShapeDtypeStruct((NP2, 128), jnp.float32),
        ],
    )(xp, degp, W1p)
    return g1, dinv


def _t2(p, g, dinv, b_pack, W_pad):
    C_in = g.shape[0]
    C_out = W_pad.shape[1] // 16
    grid = (NP2 // _PBLK,)

    def body(p_ref, g_ref, dinv_ref, b_ref, w_ref, out_ref):
        dinv = dinv_ref[...]
        w = w_ref[...]
        xt = [jnp.maximum((p_ref[0, cch] + p_ref[1, cch] + g_ref[cch]) * dinv
                          + b_ref[cch], 0.0)
              for cch in range(C_in)]
        hs = []
        for j in range(8):
            xrow = jnp.concatenate(
                [xc[:, j * 16:(j + 1) * 16] for xc in xt], axis=1)
            hs.append(jnp.dot(xrow, w, preferred_element_type=jnp.float32))
        for cch in range(C_out):
            hp = jnp.concatenate(
                [h[:, cch * 16:(cch + 1) * 16] for h in hs], axis=1)
            out_ref[cch] = hp * dinv

    return pl.pallas_call(
        body,
        grid=grid,
        in_specs=[
            pl.BlockSpec((NC, C_in, _PBLK, 128), lambda i: (0, 0, i, 0)),
            pl.BlockSpec((C_in, _PBLK, 128), lambda i: (0, i, 0)),
            pl.BlockSpec((_PBLK, 128), lambda i: (i, 0)),
            pl.BlockSpec(b_pack.shape, lambda i: (0, 0, 0)),
            pl.BlockSpec(W_pad.shape, lambda i: (0, 0)),
        ],
        out_specs=pl.BlockSpec((C_out, _PBLK, 128), lambda i: (0, i, 0)),
        out_shape=jax.ShapeDtypeStruct((C_out, NP2, 128), jnp.float32),
    )(p, g, dinv, b_pack, W_pad)


def _t3(p, g, dinv, b_pack, Wf1p, bf1, Wf2, bf2):
    C_in = g.shape[0]
    grid = (NP2 // _PBLK,)

    def body(p_ref, g_ref, dinv_ref, b_ref, w1_ref, bf1_ref, w2_ref, bf2_ref,
             out_ref):
        dinv = dinv_ref[...]
        xt = [jnp.maximum((p_ref[0, cch] + p_ref[1, cch] + g_ref[cch]) * dinv
                          + b_ref[cch], 0.0)
              for cch in range(C_in)]
        w1 = w1_ref[...]
        w2 = w2_ref[...]
        for j in range(8):
            xrow = jnp.concatenate(
                [xc[:, j * 16:(j + 1) * 16] for xc in xt], axis=1)
            t = jnp.dot(xrow, w1, preferred_element_type=jnp.float32)
            t = jnp.maximum(t + bf1_ref[...], 0.0)
            o = jnp.dot(t, w2, preferred_element_type=jnp.float32)
            out_ref[:, j] = jnp.maximum(o + bf2_ref[...], 0.0)

    return pl.pallas_call(
        body,
        grid=grid,
        in_specs=[
            pl.BlockSpec((NC, C_in, _PBLK, 128), lambda i: (0, 0, i, 0)),
            pl.BlockSpec((C_in, _PBLK, 128), lambda i: (0, i, 0)),
            pl.BlockSpec((_PBLK, 128), lambda i: (i, 0)),
            pl.BlockSpec(b_pack.shape, lambda i: (0, 0, 0)),
            pl.BlockSpec(Wf1p.shape, lambda i: (0, 0)),
            pl.BlockSpec((1, 1024), lambda i: (0, 0)),
            pl.BlockSpec(Wf2.shape, lambda i: (0, 0)),
            pl.BlockSpec((1, 128), lambda i: (0, 0)),
        ],
        out_specs=pl.BlockSpec((_PBLK, 8, 128), lambda i: (i, 0, 0)),
        out_shape=jax.ShapeDtypeStruct((NP2, 8, 128), jnp.float32),
    )(p, g, dinv, b_pack, Wf1p, bf1, Wf2, bf2)


def _pos_encoding(length, d_model):
    position = jnp.arange(length, dtype=jnp.float32)[:, None]
    div_term = jnp.exp(jnp.arange(0, d_model, 2).astype(jnp.float32)
                       * (-math.log(10000.0) / d_model))
    ang = position * div_term
    return jnp.stack([jnp.sin(ang), jnp.cos(ang)], axis=2).reshape(length, d_model)


def _pad2(w, rows, cols):
    out = jnp.zeros((rows, cols), jnp.float32)
    return out.at[: w.shape[0], : w.shape[1]].set(w)


def _bias_pack(b, C):
    bp = _pad2(b[None, :], 1, C * 16).reshape(C, 1, 16)
    return jnp.tile(bp, (1, 1, 8)).reshape(C, 1, 128)


def _tables16(g_packed):
    return [g_packed[ch].reshape(N_PAD, 16) for ch in range(g_packed.shape[0])]


def _packedNP(p):
    return p.reshape(NC, p.shape[1], N_PAD // 8, 128)


_scatter2 = _make_scatter_sc(2)
_scatter4 = _make_scatter_sc(4)
_scatter7 = _make_scatter_sc(7)


def kernel(target_x, target_edge_index, W1, b1, W2, b2, W3, b3, Wf1, bf1, Wf2, bf2):
    ei = target_edge_index.astype(jnp.int32)
    ndum = E_PAD - E
    dum = N + jnp.arange(ndum, dtype=jnp.int32) % (N_PAD - N)
    src = jnp.concatenate([ei[0], dum])
    dst = jnp.concatenate([ei[1], dum])
    idx = jnp.stack([src.reshape(-1, EB), dst.reshape(-1, EB)], axis=1)
    pe = _pos_encoding(N, IN_DIM)
    xv = jnp.zeros((N_PAD, IN_DIM), jnp.float32).at[:N].set(target_x + pe)
    xp = xv.reshape(NP2, 8 * IN_DIM)
    zeros16 = jnp.zeros((ZROWS, 16), jnp.float32)
    ones16 = jnp.ones((EB, 16), jnp.float32)

    W1p = _pad2(W1, IN_DIM, 32)
    W2p = _pad2(W2, 32, 64)
    W3p = _pad2(W3, 64, 112)
    Wf1p = _pad2(Wf1, 112, 1024)
    b1p = _bias_pack(b1, 2)
    b2p = _bias_pack(b2, 4)
    b3p = _bias_pack(b3, 7)
    bf1r = bf1[None, :]
    bf2r = bf2[None, :]

    degp = _deg_sc(dst, ones16, zeros16)
    g1, dinv = _t1(xp, degp.reshape(NC, N_PAD // 8, 128), W1p)

    p1 = _scatter2(idx, *_tables16(g1), zeros16)
    g2 = _t2(_packedNP(p1), g1, dinv, b1p, W2p)

    p2 = _scatter4(idx, *_tables16(g2), zeros16)
    g3 = _t2(_packedNP(p2), g2, dinv, b2p, W3p)

    p3 = _scatter7(idx, *_tables16(g3), zeros16)
    out = _t3(_packedNP(p3), g3, dinv, b3p, Wf1p, bf1r, Wf2, bf2r)
    return out.reshape(N_PAD, 128)[:N][None]


# speedup vs baseline: 21.5526x; 1.0063x over previous
import functools
import math

import jax
import jax.numpy as jnp
from jax import lax
from jax.experimental import pallas as pl
from jax.experimental.pallas import tpu as pltpu
from jax.experimental.pallas import tpu_sc as plsc

N = 100000
E = 1600000
IN_DIM = 26

NC = 2
NS = 16
ROWS_PER_TILE = 6400
N_PAD = NS * ROWS_PER_TILE
EB = 640
NBATCH = 80
EDGES_PER_TILE = EB * NBATCH
EDGES_PER_SC = EDGES_PER_TILE * NS
E_PAD = EDGES_PER_SC * NC
ZROWS = 320

_MESH = dict(core_axis_name="c", subcore_axis_name="s")


_SC_PARAMS = pltpu.CompilerParams(use_tc_tiling_on_sc=False)


@functools.partial(
    pl.kernel,
    mesh=plsc.VectorSubcoreMesh(**_MESH),
    out_type=jax.ShapeDtypeStruct((NC, N_PAD, 16), jnp.float32),
    compiler_params=_SC_PARAMS,
    scratch_types=[
        pltpu.VMEM((EB,), jnp.int32),
        pltpu.VMEM((EB, 16), jnp.float32),
        pltpu.VMEM((ZROWS, 16), jnp.float32),
        pltpu.VMEM_SHARED((N_PAD, 16), jnp.float32),
    ],
)
def _deg_sc(dst_hbm, ones_hbm, zeros_hbm, out_hbm, dstv, onesv, zbuf, acc):
    c = lax.axis_index("c")
    s = lax.axis_index("s")
    row0 = s * ROWS_PER_TILE
    pltpu.sync_copy(zeros_hbm, zbuf)
    pltpu.sync_copy(ones_hbm, onesv)
    for k in range(ROWS_PER_TILE // ZROWS):
        pltpu.sync_copy(zbuf, acc.at[pl.ds(row0 + k * ZROWS, ZROWS)])
    plsc.subcore_barrier()
    ebase = c * EDGES_PER_SC + s * EDGES_PER_TILE

    def body(i, carry):
        pltpu.sync_copy(dst_hbm.at[pl.ds(ebase + i * EB, EB)], dstv)
        pltpu.sync_copy(onesv, acc.at[dstv], add=True)
        return carry

    lax.fori_loop(0, NBATCH, body, 0)
    plsc.subcore_barrier()
    pltpu.sync_copy(acc.at[pl.ds(row0, ROWS_PER_TILE)],
                    out_hbm.at[c, pl.ds(row0, ROWS_PER_TILE)])


def _make_scatter_sc(C):
    @functools.partial(
        pl.kernel,
        mesh=plsc.VectorSubcoreMesh(**_MESH),
        out_type=jax.ShapeDtypeStruct((NC, C, N_PAD, 16), jnp.float32),
        compiler_params=_SC_PARAMS,
        scratch_types=[
            pltpu.VMEM((2, EB), jnp.int32),
            pltpu.VMEM((2, EB), jnp.int32),
            pltpu.VMEM((EB, 16), jnp.float32),
            pltpu.VMEM((EB, 16), jnp.float32),
            pltpu.VMEM((ZROWS, 16), jnp.float32),
            pltpu.VMEM_SHARED((N_PAD, 16), jnp.float32),
            pltpu.SemaphoreType.DMA,
            pltpu.SemaphoreType.DMA,
        ],
    )
    def scatter_sc(idx_hbm, *rest):
        tables = rest[:C]
        zeros_hbm = rest[C]
        out_hbm = rest[C + 1]
        (idx0, idx1, rows0, rows1, zbuf, acc, sem0, sem1) = rest[C + 2:]
        c = lax.axis_index("c")
        s = lax.axis_index("s")
        row0 = s * ROWS_PER_TILE
        bbase = (c * NS + s) * NBATCH
        pltpu.sync_copy(zeros_hbm, zbuf)
        for k in range(ROWS_PER_TILE // ZROWS):
            pltpu.sync_copy(zbuf, acc.at[pl.ds(row0 + k * ZROWS, ZROWS)])
        plsc.subcore_barrier()

        for ch in range(C):
            tab = tables[ch]

            pltpu.sync_copy(idx_hbm.at[bbase], idx0)
            pltpu.async_copy(tab.at[idx0.at[0]], rows0, sem0)
            pltpu.sync_copy(idx_hbm.at[bbase + 1], idx1)
            pltpu.async_copy(tab.at[idx1.at[0]], rows1, sem1)

            def body(i, carry):
                j = bbase + 2 * i
                pltpu.make_async_copy(tab.at[idx0.at[0]], rows0, sem0).wait()
                pltpu.sync_copy(rows0, acc.at[idx0.at[1]], add=True)
                pltpu.sync_copy(idx_hbm.at[j + 2], idx0)
                pltpu.async_copy(tab.at[idx0.at[0]], rows0, sem0)
                pltpu.make_async_copy(tab.at[idx1.at[0]], rows1, sem1).wait()
                pltpu.sync_copy(rows1, acc.at[idx1.at[1]], add=True)
                pltpu.sync_copy(idx_hbm.at[j + 3], idx1)
                pltpu.async_copy(tab.at[idx1.at[0]], rows1, sem1)
                return carry

            lax.fori_loop(0, (NBATCH - 2) // 2, body, 0)
            pltpu.make_async_copy(tab.at[idx0.at[0]], rows0, sem0).wait()
            pltpu.sync_copy(rows0, acc.at[idx0.at[1]], add=True)
            pltpu.make_async_copy(tab.at[idx1.at[0]], rows1, sem1).wait()
            pltpu.sync_copy(rows1, acc.at[idx1.at[1]], add=True)

            plsc.subcore_barrier()
            pltpu.sync_copy(acc.at[pl.ds(row0, ROWS_PER_TILE)],
                            out_hbm.at[c, ch, pl.ds(row0, ROWS_PER_TILE)])
            if ch < C - 1:
                for k in range(ROWS_PER_TILE // ZROWS):
                    pltpu.sync_copy(zbuf, acc.at[pl.ds(row0 + k * ZROWS, ZROWS)])
                plsc.subcore_barrier()

    return scatter_sc


_BLK = 2048
_PBLK = _BLK // 8
NP2 = N_PAD // 8


def _t1(xp, degp, W1p):
    C_out = W1p.shape[1] // 16
    grid = (NP2 // _PBLK,)

    def body(x_ref, d_ref, w_ref, g_ref, dinv_ref):
        dinv = lax.rsqrt(d_ref[0] + d_ref[1] + 1.0)
        w = w_ref[...]
        hs = [jnp.dot(x_ref[:, j * IN_DIM:(j + 1) * IN_DIM], w,
                      preferred_element_type=jnp.float32)
              for j in range(8)]
        for cch in range(C_out):
            hp = jnp.concatenate(
                [h[:, cch * 16:(cch + 1) * 16] for h in hs], axis=1)
            g_ref[cch] = hp * dinv
        dinv_ref[...] = dinv

    g1, dinv = pl.pallas_call(
        body,
        grid=grid,
        in_specs=[
            pl.BlockSpec((_PBLK, 8 * IN_DIM), lambda i: (i, 0)),
            pl.BlockSpec((NC, _PBLK, 128), lambda i: (0, i, 0)),
            pl.BlockSpec(W1p.shape, lambda i: (0, 0)),
        ],
        out_specs=[
            pl.BlockSpec((C_out, _PBLK, 128), lambda i: (0, i, 0)),
            pl.BlockSpec((_PBLK, 128), lambda i: (i, 0)),
        ],
        out_shape=[
            jax.ShapeDtypeStruct((C_out, NP2, 128), jnp.float32),
            jax.ShapeDtypeStruct((NP2, 128), jnp.float32),
        ],
    )(xp, degp, W1p)
    return g1, dinv


def _t2(p, g, dinv, b_pack, W_pad):
    C_in = g.shape[0]
    C_out = W_pad.shape[1] // 16
    grid = (NP2 // _PBLK,)

    def body(p_ref, g_ref, dinv_ref, b_ref, w_ref, out_ref):
        dinv = dinv_ref[...]
        w = w_ref[...]
        xt = [jnp.maximum((p_ref[0, cch] + p_ref[1, cch] + g_ref[cch]) * dinv
                          + b_ref[cch], 0.0)
              for cch in range(C_in)]
        hs = []
        for j in range(8):
            xrow = jnp.concatenate(
                [xc[:, j * 16:(j + 1) * 16] for xc in xt], axis=1)
            hs.append(jnp.dot(xrow, w, preferred_element_type=jnp.float32))
        for cch in range(C_out):
            hp = jnp.concatenate(
                [h[:, cch * 16:(cch + 1) * 16] for h in hs], axis=1)
            out_ref[cch] = hp * dinv

    return pl.pallas_call(
        body,
        grid=grid,
        in_specs=[
            pl.BlockSpec((NC, C_in, _PBLK, 128), lambda i: (0, 0, i, 0)),
            pl.BlockSpec((C_in, _PBLK, 128), lambda i: (0, i, 0)),
            pl.BlockSpec((_PBLK, 128), lambda i: (i, 0)),
            pl.BlockSpec(b_pack.shape, lambda i: (0, 0, 0)),
            pl.BlockSpec(W_pad.shape, lambda i: (0, 0)),
        ],
        out_specs=pl.BlockSpec((C_out, _PBLK, 128), lambda i: (0, i, 0)),
        out_shape=jax.ShapeDtypeStruct((C_out, NP2, 128), jnp.float32),
    )(p, g, dinv, b_pack, W_pad)


def _t3(p, g, dinv, b_pack, Wf1p, bf1, Wf2, bf2):
    C_in = g.shape[0]
    grid = (NP2 // _PBLK,)

    def body(p_ref, g_ref, dinv_ref, b_ref, w1_ref, bf1_ref, w2_ref, bf2_ref,
             out_ref):
        dinv = dinv_ref[...]
        xt = [jnp.maximum((p_ref[0, cch] + p_ref[1, cch] + g_ref[cch]) * dinv
                          + b_ref[cch], 0.0)
              for cch in range(C_in)]
        w1 = w1_ref[...]
        w2 = w2_ref[...]
        for j in range(8):
            xrow = jnp.concatenate(
                [xc[:, j * 16:(j + 1) * 16] for xc in xt], axis=1)
            t = jnp.dot(xrow, w1, preferred_element_type=jnp.float32)
            t = jnp.maximum(t + bf1_ref[...], 0.0)
            o = jnp.dot(t, w2, preferred_element_type=jnp.float32)
            out_ref[:, j] = jnp.maximum(o + bf2_ref[...], 0.0)

    return pl.pallas_call(
        body,
        grid=grid,
        in_specs=[
            pl.BlockSpec((NC, C_in, _PBLK, 128), lambda i: (0, 0, i, 0)),
            pl.BlockSpec((C_in, _PBLK, 128), lambda i: (0, i, 0)),
            pl.BlockSpec((_PBLK, 128), lambda i: (i, 0)),
            pl.BlockSpec(b_pack.shape, lambda i: (0, 0, 0)),
            pl.BlockSpec(Wf1p.shape, lambda i: (0, 0)),
            pl.BlockSpec((1, 1024), lambda i: (0, 0)),
            pl.BlockSpec(Wf2.shape, lambda i: (0, 0)),
            pl.BlockSpec((1, 128), lambda i: (0, 0)),
        ],
        out_specs=pl.BlockSpec((_PBLK, 8, 128), lambda i: (i, 0, 0)),
        out_shape=jax.ShapeDtypeStruct((NP2, 8, 128), jnp.float32),
    )(p, g, dinv, b_pack, Wf1p, bf1, Wf2, bf2)


def _pos_encoding(length, d_model):
    position = jnp.arange(length, dtype=jnp.float32)[:, None]
    div_term = jnp.exp(jnp.arange(0, d_model, 2).astype(jnp.float32)
                       * (-math.log(10000.0) / d_model))
    ang = position * div_term
    return jnp.stack([jnp.sin(ang), jnp.cos(ang)], axis=2).reshape(length, d_model)


def _pad2(w, rows, cols):
    out = jnp.zeros((rows, cols), jnp.float32)
    return out.at[: w.shape[0], : w.shape[1]].set(w)


def _bias_pack(b, C):
    bp = _pad2(b[None, :], 1, C * 16).reshape(C, 1, 16)
    return jnp.tile(bp, (1, 1, 8)).reshape(C, 1, 128)


def _tables16(g_packed):
    return [g_packed[ch].reshape(N_PAD, 16) for ch in range(g_packed.shape[0])]


def _packedNP(p):
    return p.reshape(NC, p.shape[1], N_PAD // 8, 128)


_scatter2 = _make_scatter_sc(2)
_scatter4 = _make_scatter_sc(4)
_scatter7 = _make_scatter_sc(7)


def kernel(target_x, target_edge_index, W1, b1, W2, b2, W3, b3, Wf1, bf1, Wf2, bf2):
    ei = target_edge_index.astype(jnp.int32)
    ndum = E_PAD - E
    dum = N + jnp.arange(ndum, dtype=jnp.int32) % (N_PAD - N)
    src = jnp.concatenate([ei[0], dum])
    dst = jnp.concatenate([ei[1], dum])
    idx = jnp.stack([src.reshape(-1, EB), dst.reshape(-1, EB)], axis=1)
    pe = _pos_encoding(N, IN_DIM)
    xv = jnp.zeros((N_PAD, IN_DIM), jnp.float32).at[:N].set(target_x + pe)
    xp = xv.reshape(NP2, 8 * IN_DIM)
    zeros16 = jnp.zeros((ZROWS, 16), jnp.float32)
    ones16 = jnp.ones((EB, 16), jnp.float32)

    W1p = _pad2(W1, IN_DIM, 32)
    W2p = _pad2(W2, 32, 64)
    W3p = _pad2(W3, 64, 112)
    Wf1p = _pad2(Wf1, 112, 1024)
    b1p = _bias_pack(b1, 2)
    b2p = _bias_pack(b2, 4)
    b3p = _bias_pack(b3, 7)
    bf1r = bf1[None, :]
    bf2r = bf2[None, :]

    degp = _deg_sc(dst, ones16, zeros16)
    g1, dinv = _t1(xp, degp.reshape(NC, N_PAD // 8, 128), W1p)

    p1 = _scatter2(idx, *_tables16(g1), zeros16)
    g2 = _t2(_packedNP(p1), g1, dinv, b1p, W2p)

    p2 = _scatter4(idx, *_tables16(g2), zeros16)
    g3 = _t2(_packedNP(p2), g2, dinv, b2p, W3p)

    p3 = _scatter7(idx, *_tables16(g3), zeros16)
    out = _t3(_packedNP(p3), g3, dinv, b3p, Wf1p, bf1r, Wf2, bf2r)
    return out.reshape(N_PAD, 128)[:N][None]
